# baseline (speedup 1.0000x reference)
import jax
import jax.numpy as jnp
from jax.experimental import pallas as pl
from jax.experimental.pallas import tpu as pltpu

NUM_HEADS = 8
D_IN = 128
GREP = 128
HEAD_DIM = GREP // NUM_HEADS
NUM_SEGMENTS = 512


def _leaky(x):
    return jnp.maximum(x, 0.01 * x)


def _block_body(nb):
    def body(x_ref, b_ref, w1c, b1c, ws2, bs2, ws3e, bs3e,
             wt2, bt2, wt3, bt3, sel, expand,
             out_ref, den_ref):
        i = pl.program_id(0)

        @pl.when(i == 0)
        def _init():
            out_ref[:] = jnp.zeros_like(out_ref)
            den_ref[:] = jnp.zeros_like(den_ref)

        x = x_ref[:]
        a1 = _leaky(jnp.dot(x, w1c[:], preferred_element_type=jnp.float32) + b1c[:])
        h = a1[:, :128]
        t = a1[:, 128:]
        h = _leaky(jnp.dot(h, ws2[:], preferred_element_type=jnp.float32) + bs2[:])
        s = jnp.dot(h, ws3e[:], preferred_element_type=jnp.float32) + bs3e[:]

        t = _leaky(jnp.dot(t, wt2[:], preferred_element_type=jnp.float32) + bt2[:])
        r = _leaky(jnp.dot(t, wt3[:], preferred_element_type=jnp.float32) + bt3[:])

        ex = jnp.exp(s)
        w = ex * r

        seg = b_ref[0, 0, :]
        blk = seg.shape[0]
        onehot_t = (jax.lax.broadcasted_iota(jnp.int32, (NUM_SEGMENTS, blk), 0)
                    == seg[None, :]).astype(jnp.float32)

        ex8 = jnp.dot(ex, sel[:], preferred_element_type=jnp.float32)
        rhs = jnp.concatenate([w, ex8], axis=1)
        upd = jnp.dot(onehot_t, rhs, preferred_element_type=jnp.float32)
        out_ref[:] += upd[:, :GREP]
        den_ref[:] += upd[:, GREP:]

        @pl.when(i == nb - 1)
        def _final():
            dexp = jnp.dot(den_ref[:], expand[:],
                           preferred_element_type=jnp.float32)
            out_ref[:] = out_ref[:] / jnp.maximum(dexp, 1e-30)

    return body


def kernel(x, batch, Ws1, bs1, Ws2, bs2, Ws3, bs3, Wt1, bt1, Wt2, bt2, Wt3, bt3):
    n = x.shape[0]
    blk = 10000 if n % 10000 == 0 else n
    nb = n // blk

    ws3e = jnp.repeat(Ws3, HEAD_DIM, axis=1)
    bs3e = jnp.repeat(bs3, HEAD_DIM).reshape(1, GREP)
    sel = (jnp.arange(GREP)[:, None] == HEAD_DIM * jnp.arange(NUM_HEADS)[None, :]
           ).astype(jnp.float32)
    expand = (jnp.arange(GREP)[None, :] // HEAD_DIM == jnp.arange(NUM_HEADS)[:, None]
              ).astype(jnp.float32)

    batch3 = batch.reshape(nb, 1, blk)
    b2 = lambda a: a.reshape(1, -1)
    w1c = jnp.concatenate([Ws1, Wt1], axis=1)
    b1c = jnp.concatenate([bs1, bt1]).reshape(1, 256)

    full = lambda shape: pl.BlockSpec(shape, lambda i: (0, 0))
    return pl.pallas_call(
        _block_body(nb),
        grid=(nb,),
        in_specs=[
            pl.BlockSpec((blk, D_IN), lambda i: (i, 0)),
            pl.BlockSpec((1, 1, blk), lambda i: (i, 0, 0)),
            full((D_IN, 256)), full((1, 256)),
            full((128, 128)), full((1, 128)),
            full((128, GREP)), full((1, GREP)),
            full((128, 128)), full((1, 128)),
            full((128, GREP)), full((1, GREP)),
            full((GREP, NUM_HEADS)), full((NUM_HEADS, GREP)),
        ],
        out_specs=pl.BlockSpec((NUM_SEGMENTS, GREP), lambda i: (0, 0)),
        out_shape=jax.ShapeDtypeStruct((NUM_SEGMENTS, GREP), jnp.float32),
        scratch_shapes=[pltpu.VMEM((NUM_SEGMENTS, NUM_HEADS), jnp.float32)],
        compiler_params=pltpu.CompilerParams(
            dimension_semantics=("arbitrary",)),
    )(x, batch3, w1c, b1c, Ws2, b2(bs2), ws3e, bs3e,
      Wt2, b2(bt2), Wt3, b2(bt3), sel, expand)


# speedup vs baseline: 1.0050x; 1.0050x over previous
import jax
import jax.numpy as jnp
from jax.experimental import pallas as pl
from jax.experimental.pallas import tpu as pltpu

NUM_HEADS = 8
D_IN = 128
GREP = 128
HEAD_DIM = GREP // NUM_HEADS
NUM_SEGMENTS = 512


def _leaky(x):
    return jnp.maximum(x, 0.01 * x)


def _block_body(nb):
    def body(x_ref, b_ref, ws1, bs1, ws2, bs2, ws3e, bs3e,
             wt1, bt1, wt2, bt2, wt3, bt3, sel, expand,
             out_ref, den_ref):
        i = pl.program_id(0)

        @pl.when(i == 0)
        def _init():
            out_ref[:] = jnp.zeros_like(out_ref)
            den_ref[:] = jnp.zeros_like(den_ref)

        x = x_ref[:]
        h = _leaky(jnp.dot(x, ws1[:], preferred_element_type=jnp.float32) + bs1[:])
        h = _leaky(jnp.dot(h, ws2[:], preferred_element_type=jnp.float32) + bs2[:])
        s = jnp.dot(h, ws3e[:], preferred_element_type=jnp.float32) + bs3e[:]

        t = _leaky(jnp.dot(x, wt1[:], preferred_element_type=jnp.float32) + bt1[:])
        t = _leaky(jnp.dot(t, wt2[:], preferred_element_type=jnp.float32) + bt2[:])
        r = _leaky(jnp.dot(t, wt3[:], preferred_element_type=jnp.float32) + bt3[:])

        ex = jnp.exp(s)
        w = ex * r

        seg = b_ref[0, 0, :]
        blk = seg.shape[0]
        onehot_t = (jax.lax.broadcasted_iota(jnp.int32, (NUM_SEGMENTS, blk), 0)
                    == seg[None, :]).astype(jnp.float32)

        ex8 = jnp.dot(ex, sel[:], preferred_element_type=jnp.float32)
        rhs = jnp.concatenate([w, ex8], axis=1)
        upd = jnp.dot(onehot_t, rhs, preferred_element_type=jnp.float32)
        out_ref[:] += upd[:, :GREP]
        den_ref[:] += upd[:, GREP:]

        @pl.when(i == nb - 1)
        def _final():
            dexp = jnp.dot(den_ref[:], expand[:],
                           preferred_element_type=jnp.float32)
            out_ref[:] = out_ref[:] / jnp.maximum(dexp, 1e-30)

    return body


def kernel(x, batch, Ws1, bs1, Ws2, bs2, Ws3, bs3, Wt1, bt1, Wt2, bt2, Wt3, bt3):
    n = x.shape[0]
    blk = 10000 if n % 10000 == 0 else n
    nb = n // blk

    ws3e = jnp.repeat(Ws3, HEAD_DIM, axis=1)
    bs3e = jnp.repeat(bs3, HEAD_DIM).reshape(1, GREP)
    sel = (jnp.arange(GREP)[:, None] == HEAD_DIM * jnp.arange(NUM_HEADS)[None, :]
           ).astype(jnp.float32)
    expand = (jnp.arange(GREP)[None, :] // HEAD_DIM == jnp.arange(NUM_HEADS)[:, None]
              ).astype(jnp.float32)

    batch3 = batch.reshape(nb, 1, blk)
    b2 = lambda a: a.reshape(1, -1)

    full = lambda shape: pl.BlockSpec(shape, lambda i: (0, 0))
    return pl.pallas_call(
        _block_body(nb),
        grid=(nb,),
        in_specs=[
            pl.BlockSpec((blk, D_IN), lambda i: (i, 0)),
            pl.BlockSpec((1, 1, blk), lambda i: (i, 0, 0)),
            full((D_IN, 128)), full((1, 128)),
            full((128, 128)), full((1, 128)),
            full((128, GREP)), full((1, GREP)),
            full((D_IN, 128)), full((1, 128)),
            full((128, 128)), full((1, 128)),
            full((128, GREP)), full((1, GREP)),
            full((GREP, NUM_HEADS)), full((NUM_HEADS, GREP)),
        ],
        out_specs=pl.BlockSpec((NUM_SEGMENTS, GREP), lambda i: (0, 0)),
        out_shape=jax.ShapeDtypeStruct((NUM_SEGMENTS, GREP), jnp.float32),
        scratch_shapes=[pltpu.VMEM((NUM_SEGMENTS, NUM_HEADS), jnp.float32)],
        compiler_params=pltpu.CompilerParams(
            dimension_semantics=("arbitrary",)),
    )(x, batch3, Ws1, b2(bs1), Ws2, b2(bs2), ws3e, bs3e,
      Wt1, b2(bt1), Wt2, b2(bt2), Wt3, b2(bt3), sel, expand)
